# baseline (speedup 1.0000x reference)
import functools

import jax
import jax.numpy as jnp
from jax import lax
from jax.experimental import pallas as pl
from jax.experimental.pallas import tpu as pltpu
from jax.experimental.pallas import tpu_sc as plsc

N = 10000
NE = 320000
F0, F1, F2 = 128, 128, 100
D = 128
NC, NS = 2, 16
NW = NC * NS
EPW = NE // NW
K = 112
CH = -(-EPW // K)
EPP = CH * K
NP = 10112
RPT = NP // NS

_mesh = plsc.VectorSubcoreMesh(core_axis_name="c", subcore_axis_name="s")


@functools.partial(
    pl.kernel,
    out_type=jax.ShapeDtypeStruct((NW * N,), jnp.float32),
    mesh=_mesh,
    scratch_types=[
        pltpu.VMEM((EPW,), jnp.int32),
        pltpu.VMEM((N,), jnp.float32),
    ],
    compiler_params=pltpu.CompilerParams(needs_layout_passes=False),
)
def _sc_degree(dst_hbm, deg_hbm, dst_v, hist_v):
    wid = lax.axis_index("s") * NC + lax.axis_index("c")
    pltpu.sync_copy(dst_hbm.at[pl.ds(wid * EPW, EPW)], dst_v)
    zeros16 = jnp.zeros((16,), jnp.float32)

    def zero_body(i, carry):
        hist_v[pl.ds(i * 16, 16)] = zeros16
        return carry

    lax.fori_loop(0, N // 16, zero_body, 0)
    ones16 = jnp.ones((16,), jnp.float32)

    def scat_body(i, carry):
        idx = dst_v[pl.ds(i * 16, 16)]
        plsc.addupdate_scatter(hist_v, [idx], ones16)
        return carry

    lax.fori_loop(0, EPW // 16, scat_body, 0)
    pltpu.sync_copy(hist_v, deg_hbm.at[pl.ds(wid * N, N)])


def _make_sc_pass(d):
    @functools.partial(
        pl.kernel,
        out_type=jax.ShapeDtypeStruct((NC, NP, d), jnp.float32),
        mesh=_mesh,
        scratch_types=[
            pltpu.VMEM((2, CH, K), jnp.int32),
            pltpu.VMEM((2, K, d), jnp.float32),
            pltpu.VMEM_SHARED((NP, d), jnp.float32),
            pltpu.SemaphoreType.DMA,
        ],
        compiler_params=pltpu.CompilerParams(use_tc_tiling_on_sc=False),
    )
    def _pass(y_hbm, edge_hbm, zero_hbm, out_hbm, e_v, rows_v, acc_sh, sem0):
        cid = lax.axis_index("c")
        sid = lax.axis_index("s")
        wid = sid * NC + cid
        row0 = sid * RPT
        pltpu.sync_copy(zero_hbm.at[pl.ds(row0, RPT)],
                        acc_sh.at[pl.ds(row0, RPT)])
        pltpu.sync_copy(edge_hbm.at[wid], e_v)
        plsc.subcore_barrier()

        pltpu.async_copy(y_hbm.at[e_v.at[0, 0]], rows_v.at[0], sem0)

        def body(g, carry):
            for b in range(2):
                ch = g * 2 + b
                nxt = lax.rem(ch + 1, CH)
                pltpu.async_copy(y_hbm.at[e_v.at[0, nxt]], rows_v.at[1 - b],
                                 sem0)
                pltpu.make_async_copy(y_hbm.at[e_v.at[0, ch]], rows_v.at[b],
                                      sem0).wait()
                pltpu.sync_copy(rows_v.at[b], acc_sh.at[e_v.at[1, ch]],
                                add=True)
            return carry

        lax.fori_loop(0, CH // 2, body, 0)
        pltpu.make_async_copy(y_hbm.at[e_v.at[0, 0]], rows_v.at[0],
                              sem0).wait()
        plsc.subcore_barrier()
        pltpu.sync_copy(acc_sh.at[pl.ds(row0, RPT)],
                        out_hbm.at[cid, pl.ds(row0, RPT)])

    return _pass


D2 = 112
_sc_pass1 = _make_sc_pass(D)
_sc_pass2 = _make_sc_pass(D2)


def _dinv(deg_ref):
    return lax.rsqrt(jnp.sum(deg_ref[...], axis=0) + 1.0)


def _tc1_body(x_ref, w_ref, deg_ref, o_ref):
    dinv = _dinv(deg_ref)
    o_ref[...] = jnp.dot(x_ref[...], w_ref[...],
                         preferred_element_type=jnp.float32) * dinv[:, None]


_tc1 = pl.pallas_call(
    _tc1_body, out_shape=jax.ShapeDtypeStruct((N, F1), jnp.float32))


def _tc2_body(p_ref, y1_ref, deg_ref, b1_ref, w2_ref, o_ref):
    dinv = _dinv(deg_ref)
    agg = p_ref[0, :N] + p_ref[1, :N] + y1_ref[...]
    h = jnp.maximum(agg * dinv[:, None] + b1_ref[...], 0.0)
    o_ref[...] = jnp.dot(h, w2_ref[...],
                         preferred_element_type=jnp.float32) * dinv[:, None]


_tc2 = pl.pallas_call(
    _tc2_body, out_shape=jax.ShapeDtypeStruct((N, D2), jnp.float32))


def _tc3_body(p_ref, y2_ref, deg_ref, b2_ref, o_ref):
    dinv = _dinv(deg_ref)
    agg = p_ref[0, :N, :F2] + p_ref[1, :N, :F2] + y2_ref[..., :F2]
    o_ref[...] = agg * dinv[:, None] + b2_ref[...]


_tc3 = pl.pallas_call(
    _tc3_body, out_shape=jax.ShapeDtypeStruct((N, F2), jnp.float32))


def kernel(x, edges, W1, b1, W2, b2):
    src_flat = edges[0].astype(jnp.int32)
    dst_flat = edges[1].astype(jnp.int32)
    pad = NW * EPP - NE
    src = jnp.concatenate(
        [src_flat, jnp.zeros((pad,), jnp.int32)]).reshape(NW, CH, K)
    dst = jnp.concatenate(
        [dst_flat,
         N + (jnp.arange(pad, dtype=jnp.int32) % (NP - N))]).reshape(NW, CH, K)
    edge = jnp.stack([src, dst], axis=1)
    deg = _sc_degree(dst_flat).reshape(NW, N)
    zeros1 = jnp.zeros((NP, D), jnp.float32)
    zeros2 = jnp.zeros((NP, D2), jnp.float32)
    y1 = _tc1(x, W1, deg)
    p1 = _sc_pass1(y1, edge, zeros1)
    W2p = jnp.pad(W2, ((0, 0), (0, D2 - F2)))
    y2 = _tc2(p1, y1, deg, b1.reshape(1, F1), W2p)
    p2 = _sc_pass2(y2, edge, zeros2)
    out = _tc3(p2, y2, deg, b2.reshape(1, F2))
    return out


# speedup vs baseline: 1.7567x; 1.7567x over previous
import functools

import jax
import jax.numpy as jnp
from jax import lax
from jax.experimental import pallas as pl
from jax.experimental.pallas import tpu as pltpu
from jax.experimental.pallas import tpu_sc as plsc

N = 10000
NE = 320000
F0, F1, F2 = 128, 128, 100
D = 128
NC, NS = 2, 16
NW = NC * NS
EPW = NE // NW
K = 100
CH = -(-EPW // K)
EPP = CH * K
NP = 10240
RPT = NP // NS

_mesh = plsc.VectorSubcoreMesh(core_axis_name="c", subcore_axis_name="s")


@functools.partial(
    pl.kernel,
    out_type=jax.ShapeDtypeStruct((NW * N,), jnp.float32),
    mesh=_mesh,
    scratch_types=[
        pltpu.VMEM((EPW,), jnp.int32),
        pltpu.VMEM((N,), jnp.float32),
    ],
    compiler_params=pltpu.CompilerParams(needs_layout_passes=False),
)
def _sc_degree(dst_hbm, deg_hbm, dst_v, hist_v):
    wid = lax.axis_index("s") * NC + lax.axis_index("c")
    pltpu.sync_copy(dst_hbm.at[pl.ds(wid * EPW, EPW)], dst_v)
    zeros16 = jnp.zeros((16,), jnp.float32)

    def zero_body(i, carry):
        hist_v[pl.ds(i * 16, 16)] = zeros16
        return carry

    lax.fori_loop(0, N // 16, zero_body, 0)
    ones16 = jnp.ones((16,), jnp.float32)

    def scat_body(i, carry):
        idx = dst_v[pl.ds(i * 16, 16)]
        plsc.addupdate_scatter(hist_v, [idx], ones16)
        return carry

    lax.fori_loop(0, EPW // 16, scat_body, 0)
    pltpu.sync_copy(hist_v, deg_hbm.at[pl.ds(wid * N, N)])


def _make_sc_pass(d):
    @functools.partial(
        pl.kernel,
        out_type=jax.ShapeDtypeStruct((NC, NP, d), jnp.float32),
        mesh=_mesh,
        scratch_types=[
            pltpu.VMEM((2, CH, K), jnp.int32),
            pltpu.VMEM((2, K, d), jnp.float32),
            pltpu.VMEM_SHARED((NP, d), jnp.float32),
            pltpu.SemaphoreType.DMA,
        ],
        compiler_params=pltpu.CompilerParams(use_tc_tiling_on_sc=False),
    )
    def _pass(y_hbm, edge_hbm, zero_hbm, out_hbm, e_v, rows_v, acc_sh, sem0):
        cid = lax.axis_index("c")
        sid = lax.axis_index("s")
        wid = sid * NC + cid
        row0 = sid * RPT
        pltpu.sync_copy(zero_hbm.at[pl.ds(row0, RPT)],
                        acc_sh.at[pl.ds(row0, RPT)])
        pltpu.sync_copy(edge_hbm.at[wid], e_v)
        plsc.subcore_barrier()

        pltpu.async_copy(y_hbm.at[e_v.at[0, 0]], rows_v.at[0], sem0)

        def body(g, carry):
            for b in range(2):
                ch = g * 2 + b
                nxt = lax.rem(ch + 1, CH)
                pltpu.async_copy(y_hbm.at[e_v.at[0, nxt]], rows_v.at[1 - b],
                                 sem0)
                pltpu.make_async_copy(y_hbm.at[e_v.at[0, ch]], rows_v.at[b],
                                      sem0).wait()
                pltpu.sync_copy(rows_v.at[b], acc_sh.at[e_v.at[1, ch]],
                                add=True)
            return carry

        lax.fori_loop(0, CH // 2, body, 0)
        pltpu.make_async_copy(y_hbm.at[e_v.at[0, 0]], rows_v.at[0],
                              sem0).wait()
        plsc.subcore_barrier()
        pltpu.sync_copy(acc_sh.at[pl.ds(row0, RPT)],
                        out_hbm.at[cid, pl.ds(row0, RPT)])

    return _pass


D2 = 112
_sc_pass1 = _make_sc_pass(D)
_sc_pass2 = _make_sc_pass(D2)


def _dinv(deg_ref):
    return lax.rsqrt(jnp.sum(deg_ref[...], axis=0) + 1.0)


def _tc1a_body(x_ref, w_ref, o_ref):
    o_ref[...] = jnp.dot(x_ref[...], w_ref[...],
                         preferred_element_type=jnp.float32)


_tc1a = pl.pallas_call(
    _tc1a_body, out_shape=jax.ShapeDtypeStruct((N, F1), jnp.float32))


def _tc1b_body(h_ref, deg_ref, o_ref):
    dinv = _dinv(deg_ref)
    o_ref[...] = h_ref[...] * dinv[:, None]


_tc1b = pl.pallas_call(
    _tc1b_body, out_shape=jax.ShapeDtypeStruct((N, F1), jnp.float32))


def _tc2_body(p_ref, y1_ref, deg_ref, b1_ref, w2_ref, o_ref):
    dinv = _dinv(deg_ref)
    agg = p_ref[0, :N] + p_ref[1, :N] + y1_ref[...]
    h = jnp.maximum(agg * dinv[:, None] + b1_ref[...], 0.0)
    o_ref[...] = jnp.dot(h, w2_ref[...],
                         preferred_element_type=jnp.float32) * dinv[:, None]


_tc2 = pl.pallas_call(
    _tc2_body, out_shape=jax.ShapeDtypeStruct((N, D2), jnp.float32))


def _tc3_body(p_ref, y2_ref, deg_ref, b2_ref, o_ref):
    dinv = _dinv(deg_ref)
    agg = p_ref[0, :N, :F2] + p_ref[1, :N, :F2] + y2_ref[..., :F2]
    o_ref[...] = agg * dinv[:, None] + b2_ref[...]


_tc3 = pl.pallas_call(
    _tc3_body, out_shape=jax.ShapeDtypeStruct((N, F2), jnp.float32))


def kernel(x, edges, W1, b1, W2, b2):
    src_flat = edges[0].astype(jnp.int32)
    dst_flat = edges[1].astype(jnp.int32)
    pad = NW * EPP - NE
    src = jnp.concatenate(
        [src_flat, jnp.zeros((pad,), jnp.int32)]).reshape(NW, CH, K)
    dst = jnp.concatenate(
        [dst_flat,
         N + (jnp.arange(pad, dtype=jnp.int32) % (NP - N))]).reshape(NW, CH, K)
    edge = jnp.stack([src, dst], axis=1)
    deg = _sc_degree(dst_flat).reshape(NW, N)
    zeros1 = jnp.zeros((NP, D), jnp.float32)
    zeros2 = jnp.zeros((NP, D2), jnp.float32)
    y1 = _tc1b(_tc1a(x, W1), deg)
    p1 = _sc_pass1(y1, edge, zeros1)
    W2p = jnp.pad(W2, ((0, 0), (0, D2 - F2)))
    y2 = _tc2(p1, y1, deg, b1.reshape(1, F1), W2p)
    p2 = _sc_pass2(y2, edge, zeros2)
    out = _tc3(p2, y2, deg, b2.reshape(1, F2))
    return out


# speedup vs baseline: 1.7996x; 1.0244x over previous
import functools

import jax
import jax.numpy as jnp
from jax import lax
from jax.experimental import pallas as pl
from jax.experimental.pallas import tpu as pltpu
from jax.experimental.pallas import tpu_sc as plsc

N = 10000
NE = 320000
F0, F1, F2 = 128, 128, 100
D = 128
NC, NS = 2, 16
NW = NC * NS
EPW = NE // NW
K = 100
CH = -(-EPW // K)
EPP = CH * K
NP = 10240
RPT = NP // NS

_mesh = plsc.VectorSubcoreMesh(core_axis_name="c", subcore_axis_name="s")


@functools.partial(
    pl.kernel,
    out_type=jax.ShapeDtypeStruct((NW * N,), jnp.float32),
    mesh=_mesh,
    scratch_types=[
        pltpu.VMEM((EPW,), jnp.int32),
        pltpu.VMEM((N,), jnp.float32),
    ],
    compiler_params=pltpu.CompilerParams(needs_layout_passes=False),
)
def _sc_degree(dst_hbm, deg_hbm, dst_v, hist_v):
    wid = lax.axis_index("s") * NC + lax.axis_index("c")
    pltpu.sync_copy(dst_hbm.at[pl.ds(wid * EPW, EPW)], dst_v)
    zeros16 = jnp.zeros((16,), jnp.float32)

    def zero_body(i, carry):
        hist_v[pl.ds(i * 16, 16)] = zeros16
        return carry

    lax.fori_loop(0, N // 16, zero_body, 0)
    ones16 = jnp.ones((16,), jnp.float32)

    def scat_body(i, carry):
        idx = dst_v[pl.ds(i * 16, 16)]
        plsc.addupdate_scatter(hist_v, [idx], ones16)
        return carry

    lax.fori_loop(0, EPW // 16, scat_body, 0)
    pltpu.sync_copy(hist_v, deg_hbm.at[pl.ds(wid * N, N)])


def _make_sc_pass(d):
    @functools.partial(
        pl.kernel,
        out_type=jax.ShapeDtypeStruct((NC, NP, d), jnp.float32),
        mesh=_mesh,
        scratch_types=[
            pltpu.VMEM((2, CH, K), jnp.int32),
            pltpu.VMEM((2, K, d), jnp.float32),
            pltpu.VMEM_SHARED((NP, d), jnp.float32),
            pltpu.SemaphoreType.DMA,
        ],
        compiler_params=pltpu.CompilerParams(use_tc_tiling_on_sc=False),
    )
    def _pass(y_hbm, edge_hbm, zero_hbm, out_hbm, e_v, rows_v, acc_sh, sem0):
        cid = lax.axis_index("c")
        sid = lax.axis_index("s")
        wid = sid * NC + cid
        row0 = sid * RPT
        pltpu.sync_copy(edge_hbm.at[wid], e_v)
        pltpu.async_copy(y_hbm.at[e_v.at[0, 0]], rows_v.at[0], sem0)
        pltpu.sync_copy(zero_hbm.at[pl.ds(row0, RPT)],
                        acc_sh.at[pl.ds(row0, RPT)])
        plsc.subcore_barrier()

        def body(g, carry):
            for b in range(2):
                ch = g * 2 + b
                nxt = lax.rem(ch + 1, CH)
                pltpu.async_copy(y_hbm.at[e_v.at[0, nxt]], rows_v.at[1 - b],
                                 sem0)
                pltpu.make_async_copy(y_hbm.at[e_v.at[0, ch]], rows_v.at[b],
                                      sem0).wait()
                pltpu.sync_copy(rows_v.at[b], acc_sh.at[e_v.at[1, ch]],
                                add=True)
            return carry

        lax.fori_loop(0, CH // 2, body, 0)
        pltpu.make_async_copy(y_hbm.at[e_v.at[0, 0]], rows_v.at[0],
                              sem0).wait()
        plsc.subcore_barrier()
        pltpu.sync_copy(acc_sh.at[pl.ds(row0, RPT)],
                        out_hbm.at[cid, pl.ds(row0, RPT)])

    return _pass


D2 = 112
_sc_pass1 = _make_sc_pass(D)
_sc_pass2 = _make_sc_pass(D2)


def _dinv(deg_ref):
    return lax.rsqrt(jnp.sum(deg_ref[...], axis=0) + 1.0)


def _tc1_body(x_ref, w_ref, deg_ref, o_ref):
    dinv = _dinv(deg_ref)
    o_ref[...] = jnp.dot(x_ref[...], w_ref[...],
                         preferred_element_type=jnp.float32) * dinv[:, None]


_tc1 = pl.pallas_call(
    _tc1_body, out_shape=jax.ShapeDtypeStruct((N, F1), jnp.float32))


def _tc2_body(p_ref, y1_ref, deg_ref, b1_ref, w2_ref, o_ref):
    dinv = _dinv(deg_ref)
    agg = p_ref[0, :N] + p_ref[1, :N] + y1_ref[...]
    h = jnp.maximum(agg * dinv[:, None] + b1_ref[...], 0.0)
    o_ref[...] = jnp.dot(h, w2_ref[...],
                         preferred_element_type=jnp.float32) * dinv[:, None]


_tc2 = pl.pallas_call(
    _tc2_body, out_shape=jax.ShapeDtypeStruct((N, D2), jnp.float32))


def _tc3_body(p_ref, y2_ref, deg_ref, b2_ref, o_ref):
    dinv = _dinv(deg_ref)
    agg = p_ref[0, :N, :F2] + p_ref[1, :N, :F2] + y2_ref[..., :F2]
    o_ref[...] = agg * dinv[:, None] + b2_ref[...]


_tc3 = pl.pallas_call(
    _tc3_body, out_shape=jax.ShapeDtypeStruct((N, F2), jnp.float32))


def kernel(x, edges, W1, b1, W2, b2):
    src_flat = edges[0].astype(jnp.int32)
    dst_flat = edges[1].astype(jnp.int32)
    pad = NW * EPP - NE
    src = jnp.concatenate(
        [src_flat, jnp.zeros((pad,), jnp.int32)]).reshape(NW, CH, K)
    dst = jnp.concatenate(
        [dst_flat,
         N + (jnp.arange(pad, dtype=jnp.int32) % (NP - N))]).reshape(NW, CH, K)
    edge = jnp.stack([src, dst], axis=1)
    deg = _sc_degree(dst_flat).reshape(NW, N)
    zeros1 = jnp.zeros((NP, D), jnp.float32)
    zeros2 = jnp.zeros((NP, D2), jnp.float32)
    y1 = _tc1(x, W1, deg)
    p1 = _sc_pass1(y1, edge, zeros1)
    W2p = jnp.pad(W2, ((0, 0), (0, D2 - F2)))
    y2 = _tc2(p1, y1, deg, b1.reshape(1, F1), W2p)
    p2 = _sc_pass2(y2, edge, zeros2)
    out = _tc3(p2, y2, deg, b2.reshape(1, F2))
    return out
